# baseline (speedup 1.0000x reference)
import functools

import jax
import jax.numpy as jnp
from jax import lax
from jax.experimental import pallas as pl
from jax.experimental.pallas import tpu as pltpu
from jax.experimental.pallas import tpu_sc as plsc

_N = 65536
_K = 512
_W = 16
_NC = 2
_NS = 16
_NW = _NC * _NS
_PPW = _N // _NW
_CHUNK = 128
_NCHUNK = _PPW // _CHUNK
_L = 16

_BP = 2048
_NB = _N // _BP


_sc_mesh = plsc.VectorSubcoreMesh(core_axis_name="c", subcore_axis_name="s")


@functools.partial(
    pl.kernel,
    out_type=jax.ShapeDtypeStruct((_NC, _K, _W), jnp.float32),
    mesh=_sc_mesh,
    scratch_types=[
        pltpu.VMEM((_PPW,), jnp.float32),
        pltpu.VMEM((_PPW,), jnp.float32),
        pltpu.VMEM((_PPW, _W), jnp.float32),
        pltpu.VMEM((_PPW,), jnp.int32),
        pltpu.VMEM((_NCHUNK, _CHUNK), jnp.int32),
        pltpu.VMEM_SHARED((_K, _W), jnp.float32),
        pltpu.SemaphoreType.DMA,
    ],
    compiler_params=pltpu.CompilerParams(
        use_tc_tiling_on_sc=False, needs_layout_passes=False
    ),
)
def _sc_segsum(vt_hbm, idx_hbm, zero_hbm, part_hbm,
               vx_v, vy_v, aug_v, idxf_v, idx_v, acc_sh, sem):
    c = lax.axis_index("c")
    s = lax.axis_index("s")
    w = c * _NS + s
    base = w * _PPW
    cp1 = pltpu.async_copy(vt_hbm.at[0, pl.ds(base, _PPW)], vx_v, sem)
    cp2 = pltpu.async_copy(vt_hbm.at[1, pl.ds(base, _PPW)], vy_v, sem)
    cp3 = pltpu.async_copy(idx_hbm.at[pl.ds(base, _PPW)], idxf_v, sem)
    cp1.wait()
    cp2.wait()
    cp3.wait()
    @pl.when(s == 0)
    def _():
        pltpu.sync_copy(zero_hbm, acc_sh)

    ids = lax.iota(jnp.int32, _L)
    col0 = jnp.zeros((_L,), jnp.int32)
    col1 = jnp.full((_L,), 1, jnp.int32)
    col2 = jnp.full((_L,), 2, jnp.int32)
    ones = jnp.full((_L,), 1.0, jnp.float32)

    def build_body(i, carry):
        rows = ids + i * _L
        plsc.store_scatter(aug_v, [rows, col0], vx_v[pl.ds(i * _L, _L)])
        plsc.store_scatter(aug_v, [rows, col1], vy_v[pl.ds(i * _L, _L)])
        plsc.store_scatter(aug_v, [rows, col2], ones)
        idx_v[lax.shift_right_logical(i, 3), pl.ds(lax.bitwise_and(i, 7) * _L, _L)] = (
            idxf_v[pl.ds(i * _L, _L)]
        )
        return carry

    lax.fori_loop(0, _PPW // _L, build_body, 0)

    plsc.subcore_barrier()
    cps = [
        pltpu.async_copy(
            aug_v.at[pl.ds(j * _CHUNK, _CHUNK)],
            acc_sh.at[idx_v.at[j]],
            sem,
            add=True,
        )
        for j in range(_NCHUNK)
    ]
    for cp in cps:
        cp.wait()
    plsc.subcore_barrier()
    @pl.when(s == 0)
    def _():
        pltpu.sync_copy(acc_sh, part_hbm.at[c])


_R = 8
_NR = _K // _R


def _tc_body(partials_ref, vt_ref, cent_ref, assign_ref, cxy_s, part_s, dsem):
    i = pl.program_id(0)

    @pl.when(i == 0)
    def _():
        pltpu.make_async_copy(partials_ref, part_s, dsem).start()
        pltpu.make_async_copy(partials_ref, part_s, dsem).wait()
        psum = part_s[0, :, :] + part_s[1, :, :]
        cnt = psum[:, 2:3]
        c01 = jnp.concatenate([psum[:, 0:1] / cnt, psum[:, 1:2] / cnt], axis=1)
        cxy_s[...] = c01
        cent_ref[...] = c01

    cx = cxy_s[:, 0:1]
    cy = cxy_s[:, 1:2]
    vx = vt_ref[0:1, :]
    vy = vt_ref[1:2, :]
    riota = lax.broadcasted_iota(jnp.int32, (_R, 1), 0).astype(jnp.float32)
    m = jnp.full((_R, _BP), jnp.inf, jnp.float32)
    bi = jnp.zeros((_R, _BP), jnp.float32)
    for j in range(_NR):
        cxj = cx[j * _R:(j + 1) * _R, :]
        cyj = cy[j * _R:(j + 1) * _R, :]
        dx = vx - cxj
        dy = vy - cyj
        d = dx * dx + dy * dy
        better = d < m
        m = jnp.where(better, d, m)
        bi = jnp.where(better, riota + jnp.float32(j * _R), bi)
    m_all = jnp.min(m, axis=0, keepdims=True)
    idx = jnp.min(jnp.where(m == m_all, bi, jnp.float32(_K)), axis=0)
    assign_ref[...] = idx.astype(jnp.int32)


_tc_assign = pl.pallas_call(
    _tc_body,
    grid=(_NB,),
    in_specs=[
        pl.BlockSpec(memory_space=pl.ANY),
        pl.BlockSpec((2, _BP), lambda i: (0, i)),
    ],
    out_specs=[
        pl.BlockSpec((_K, 2), lambda i: (0, 0)),
        pl.BlockSpec((_BP,), lambda i: (i,)),
    ],
    out_shape=[
        jax.ShapeDtypeStruct((_K, 2), jnp.float32),
        jax.ShapeDtypeStruct((_N,), jnp.int32),
    ],
    scratch_shapes=[
        pltpu.VMEM((_K, 2), jnp.float32),
        pltpu.VMEM((_NC, _K, _W), jnp.float32),
        pltpu.SemaphoreType.DMA,
    ],
)


def kernel(vectors, centroids, assignment):
    del centroids
    zero = jnp.zeros((_K, _W), jnp.float32)
    vt = vectors.T
    partials = _sc_segsum(vt, assignment, zero)
    cent, a2 = _tc_assign(partials, vt)
    return cent, a2


# speedup vs baseline: 1.0115x; 1.0115x over previous
import functools

import jax
import jax.numpy as jnp
from jax import lax
from jax.experimental import pallas as pl
from jax.experimental.pallas import tpu as pltpu
from jax.experimental.pallas import tpu_sc as plsc

_N = 65536
_K = 512
_W = 16
_NC = 2
_NS = 16
_NW = _NC * _NS
_PPW = _N // _NW
_CHUNK = 128
_NCHUNK = _PPW // _CHUNK
_L = 16

_BP = 2048
_NB = _N // _BP


_sc_mesh = plsc.VectorSubcoreMesh(core_axis_name="c", subcore_axis_name="s")


@functools.partial(
    pl.kernel,
    out_type=jax.ShapeDtypeStruct((_NC, _K, _W), jnp.float32),
    mesh=_sc_mesh,
    scratch_types=[
        pltpu.VMEM((_PPW,), jnp.float32),
        pltpu.VMEM((_PPW,), jnp.float32),
        pltpu.VMEM((_PPW, _W), jnp.float32),
        pltpu.VMEM((_PPW,), jnp.int32),
        pltpu.VMEM((_NCHUNK, _CHUNK), jnp.int32),
        pltpu.VMEM_SHARED((_K, _W), jnp.float32),
        pltpu.SemaphoreType.DMA,
    ],
    compiler_params=pltpu.CompilerParams(
        use_tc_tiling_on_sc=False, needs_layout_passes=False
    ),
)
def _sc_segsum(vt_hbm, idx_hbm, zero_hbm, part_hbm,
               vx_v, vy_v, aug_v, idxf_v, idx_v, acc_sh, sem):
    c = lax.axis_index("c")
    s = lax.axis_index("s")
    w = c * _NS + s
    base = w * _PPW
    cp1 = pltpu.async_copy(vt_hbm.at[0, pl.ds(base, _PPW)], vx_v, sem)
    cp2 = pltpu.async_copy(vt_hbm.at[1, pl.ds(base, _PPW)], vy_v, sem)
    cp3 = pltpu.async_copy(idx_hbm.at[pl.ds(base, _PPW)], idxf_v, sem)
    cp1.wait()
    cp2.wait()
    cp3.wait()
    @pl.when(s == 0)
    def _():
        pltpu.sync_copy(zero_hbm, acc_sh)

    ids = lax.iota(jnp.int32, _L)
    col0 = jnp.zeros((_L,), jnp.int32)
    col1 = jnp.full((_L,), 1, jnp.int32)
    col2 = jnp.full((_L,), 2, jnp.int32)
    ones = jnp.full((_L,), 1.0, jnp.float32)

    def build_body(i, carry):
        rows = ids + i * _L
        plsc.store_scatter(aug_v, [rows, col0], vx_v[pl.ds(i * _L, _L)])
        plsc.store_scatter(aug_v, [rows, col1], vy_v[pl.ds(i * _L, _L)])
        plsc.store_scatter(aug_v, [rows, col2], ones)
        idx_v[lax.shift_right_logical(i, 3), pl.ds(lax.bitwise_and(i, 7) * _L, _L)] = (
            idxf_v[pl.ds(i * _L, _L)]
        )
        return carry

    lax.fori_loop(0, _PPW // _L, build_body, 0)

    plsc.subcore_barrier()
    cps = [
        pltpu.async_copy(
            aug_v.at[pl.ds(j * _CHUNK, _CHUNK)],
            acc_sh.at[idx_v.at[j]],
            sem,
            add=True,
        )
        for j in range(_NCHUNK)
    ]
    for cp in cps:
        cp.wait()
    plsc.subcore_barrier()
    @pl.when(s == 0)
    def _():
        pltpu.sync_copy(acc_sh, part_hbm.at[c])


_R = 8
_NR = _K // _R


def _tc_body(partials_ref, vt_ref, cent_ref, assign_ref, cxy_s):
    i = pl.program_id(0)

    @pl.when(i == 0)
    def _():
        psum = partials_ref[0, :, :] + partials_ref[1, :, :]
        cnt = psum[:, 2:3]
        c01 = jnp.concatenate([psum[:, 0:1] / cnt, psum[:, 1:2] / cnt], axis=1)
        cxy_s[...] = c01
        cent_ref[...] = c01

    cx = cxy_s[:, 0:1]
    cy = cxy_s[:, 1:2]
    vx = vt_ref[0:1, :]
    vy = vt_ref[1:2, :]
    riota = lax.broadcasted_iota(jnp.int32, (_R, 1), 0).astype(jnp.float32)
    m = jnp.full((_R, _BP), jnp.inf, jnp.float32)
    bi = jnp.zeros((_R, _BP), jnp.float32)
    for j in range(_NR):
        cxj = cx[j * _R:(j + 1) * _R, :]
        cyj = cy[j * _R:(j + 1) * _R, :]
        dx = vx - cxj
        dy = vy - cyj
        d = dx * dx + dy * dy
        better = d < m
        m = jnp.where(better, d, m)
        bi = jnp.where(better, riota + jnp.float32(j * _R), bi)
    m_all = jnp.min(m, axis=0, keepdims=True)
    idx = jnp.min(jnp.where(m == m_all, bi, jnp.float32(_K)), axis=0)
    assign_ref[...] = idx.astype(jnp.int32)


_tc_assign = pl.pallas_call(
    _tc_body,
    grid=(_NB,),
    in_specs=[
        pl.BlockSpec((_NC, _K, _W), lambda i: (0, 0, 0)),
        pl.BlockSpec((2, _BP), lambda i: (0, i)),
    ],
    out_specs=[
        pl.BlockSpec((_K, 2), lambda i: (0, 0)),
        pl.BlockSpec((_BP,), lambda i: (i,)),
    ],
    out_shape=[
        jax.ShapeDtypeStruct((_K, 2), jnp.float32),
        jax.ShapeDtypeStruct((_N,), jnp.int32),
    ],
    scratch_shapes=[pltpu.VMEM((_K, 2), jnp.float32)],
)


def kernel(vectors, centroids, assignment):
    del centroids
    zero = jnp.zeros((_K, _W), jnp.float32)
    vt = vectors.T
    partials = _sc_segsum(vt, assignment, zero)
    cent, a2 = _tc_assign(partials, vt)
    return cent, a2
